# baseline (speedup 1.0000x reference)
import jax
import jax.numpy as jnp
from jax.experimental import pallas as pl
from jax.experimental.pallas import tpu as pltpu


def _conv_rows_kernel(x_ref, w_ref, b_ref, o_ref):
    for i in range(x_ref.shape[0]):
        acc = jnp.dot(w_ref[...], x_ref[i], preferred_element_type=jnp.float32)
        o_ref[i] = (acc + b_ref[...]).astype(o_ref.dtype)


def kernel(x, weight, bias):
    N, C_in, L = x.shape
    C_out = weight.shape[0]
    R = 2 if N % 2 == 0 else 1

    w2d = weight[:, :, 0]
    b2d = bias.reshape(C_out, 1)

    grid = (N // R,)
    return pl.pallas_call(
        _conv_rows_kernel,
        out_shape=jax.ShapeDtypeStruct((N, C_out, L), x.dtype),
        grid=grid,
        in_specs=[
            pl.BlockSpec((R, C_in, L), lambda n: (n, 0, 0)),
            pl.BlockSpec((C_out, C_in), lambda n: (0, 0)),
            pl.BlockSpec((C_out, 1), lambda n: (0, 0)),
        ],
        out_specs=pl.BlockSpec((R, C_out, L), lambda n: (n, 0, 0)),
        compiler_params=pltpu.CompilerParams(
            dimension_semantics=("parallel",),
            vmem_limit_bytes=40 * 1024 * 1024,
        ),
    )(x, w2d, b2d)
